# baseline (speedup 1.0000x reference)
import functools

import jax
import jax.numpy as jnp
from jax import lax
from jax.experimental import pallas as pl
from jax.experimental.pallas import tpu as pltpu
from jax.experimental.pallas import tpu_sc as plsc

D = 64
K = 1024
B = 32
T = 576
GRP = 8
N = B * T

_COMMIT = 0.25

_NW = 32
_BPW = N // _NW
_CHUNK = 128
_NFULL = _BPW // _CHUNK
_TAIL = _BPW - _NFULL * _CHUNK


def _tc_body(xt_ref, e_ref, idx_ref, loss_ref, perp_ref, counts_acc, e2col, sse_acc):
    step = pl.program_id(0)

    e = e_ref[...]

    @pl.when(step == 0)
    def _init():
        counts_acc[...] = jnp.zeros_like(counts_acc)
        sse_acc[...] = jnp.zeros_like(sse_acc)
        e2 = jnp.sum(e * e, axis=0, keepdims=True)
        e2col[...] = e2.reshape(K, 1)

    e2c = e2col[...]
    for g in range(GRP):
        xt = xt_ref[g]
        s2 = lax.dot_general(
            e, xt * jnp.float32(-2.0),
            (((0,), (0,)), ((), ())),
            preferred_element_type=jnp.float32,
        )
        x2 = jnp.sum(xt * xt, axis=0, keepdims=True)
        d = (x2 + s2) + e2c

        dmin = jnp.min(d, axis=0, keepdims=True)
        iota = lax.broadcasted_iota(jnp.int32, (K, T), 0).astype(jnp.float32)
        masked = jnp.where(d == dmin, iota, jnp.float32(K))
        idxf = jnp.min(masked, axis=0, keepdims=True)
        idx_ref[g] = idxf.astype(jnp.int32)

        onehot = jnp.where(masked == idxf, jnp.float32(1.0), jnp.float32(0.0))
        counts_acc[...] += onehot
        sse_acc[...] += dmin

    @pl.when(step == pl.num_programs(0) - 1)
    def _finish():
        counts = jnp.sum(counts_acc[...], axis=1, keepdims=True)
        p = counts / jnp.float32(N)
        ent = -jnp.sum(p * jnp.log(p + 1e-10))
        perp_ref[0, 0] = jnp.exp(ent)
        loss_ref[0, 0] = (1.0 + _COMMIT) * jnp.sum(sse_acc[...]) / jnp.float32(N * D)


_tc_call = pl.pallas_call(
    _tc_body,
    grid=(B // GRP,),
    in_specs=[
        pl.BlockSpec((GRP, D, T), lambda i: (i, 0, 0)),
        pl.BlockSpec((D, K), lambda i: (0, 0)),
    ],
    out_specs=[
        pl.BlockSpec((GRP, 1, T), lambda i: (i, 0, 0)),
        pl.BlockSpec(memory_space=pltpu.SMEM, block_shape=(1, 1), index_map=lambda i: (0, 0)),
        pl.BlockSpec(memory_space=pltpu.SMEM, block_shape=(1, 1), index_map=lambda i: (0, 0)),
    ],
    out_shape=[
        jax.ShapeDtypeStruct((B, 1, T), jnp.int32),
        jax.ShapeDtypeStruct((1, 1), jnp.float32),
        jax.ShapeDtypeStruct((1, 1), jnp.float32),
    ],
    scratch_shapes=[
        pltpu.VMEM((K, T), jnp.float32),
        pltpu.VMEM((K, 1), jnp.float32),
        pltpu.VMEM((1, T), jnp.float32),
    ],
)


@functools.lru_cache(maxsize=None)
def _make_sc_gather():
    @functools.partial(
        pl.kernel,
        mesh=plsc.VectorSubcoreMesh(core_axis_name="c", subcore_axis_name="s"),
        out_type=jax.ShapeDtypeStruct((B, T, D), jnp.float32),
        scratch_types=[
            pltpu.VMEM((_BPW,), jnp.int32),
            pltpu.VMEM((_BPW, D), jnp.float32),
            pltpu.SemaphoreType.DMA,
        ],
        compiler_params=pltpu.CompilerParams(use_tc_tiling_on_sc=False),
    )
    def _sc_gather(table_hbm, idx_hbm, out_hbm, idx_v, rows_v, sem):
        wid = lax.axis_index("s") * 2 + lax.axis_index("c")
        pltpu.sync_copy(idx_hbm.at[wid, 0], idx_v)
        copies = []
        for j in range(_NFULL):
            copies.append(
                pltpu.async_copy(
                    table_hbm.at[idx_v.at[pl.ds(j * _CHUNK, _CHUNK)]],
                    rows_v.at[pl.ds(j * _CHUNK, _CHUNK)],
                    sem,
                )
            )
        copies.append(
            pltpu.async_copy(
                table_hbm.at[idx_v.at[pl.ds(_NFULL * _CHUNK, _TAIL)]],
                rows_v.at[pl.ds(_NFULL * _CHUNK, _TAIL)],
                sem,
            )
        )
        for c in copies:
            c.wait()
        pltpu.sync_copy(rows_v, out_hbm.at[wid])

    return _sc_gather


def kernel(inputs, embeddings):
    xt = jnp.transpose(inputs, (0, 2, 1))
    idx, loss, perp = _tc_call(xt, embeddings)
    table = embeddings.T
    quantized_st = _make_sc_gather()(table, idx)
    return quantized_st, loss.reshape(()), perp.reshape(())


# speedup vs baseline: 1.0021x; 1.0021x over previous
import functools

import jax
import jax.numpy as jnp
from jax import lax
from jax.experimental import pallas as pl
from jax.experimental.pallas import tpu as pltpu
from jax.experimental.pallas import tpu_sc as plsc

D = 64
K = 1024
B = 32
T = 576
GRP = 8
TPAD = 640
N = B * T

_COMMIT = 0.25

_NW = 32
_BPW = N // _NW
_CHUNK = 128
_NFULL = _BPW // _CHUNK
_TAIL = _BPW - _NFULL * _CHUNK


def _tc_body(xt_ref, e_ref, idx_ref, loss_ref, perp_ref, counts_acc, e2col, sse_acc):
    step = pl.program_id(0)

    e = e_ref[...]

    @pl.when(step == 0)
    def _init():
        counts_acc[...] = jnp.zeros_like(counts_acc)
        sse_acc[...] = jnp.zeros_like(sse_acc)
        e2 = jnp.sum(e * e, axis=0, keepdims=True)
        e2col[...] = e2.reshape(K, 1)

    e2c = e2col[...]
    for g in range(GRP):
        xt = xt_ref[g]
        s2 = lax.dot_general(
            e, xt * jnp.float32(-2.0),
            (((0,), (0,)), ((), ())),
            preferred_element_type=jnp.float32,
        )
        x2 = jnp.sum(xt * xt, axis=0, keepdims=True)
        d = (x2 + s2) + e2c

        dmin = jnp.min(d, axis=0, keepdims=True)
        iota = lax.broadcasted_iota(jnp.int32, (K, T), 0).astype(jnp.float32)
        masked = jnp.where(d == dmin, iota, jnp.float32(K))
        idxf = jnp.min(masked, axis=0, keepdims=True)
        idx_ref[g, 0, pl.ds(0, T)] = idxf.astype(jnp.int32).reshape(T)

        onehot = jnp.where(masked == idxf, jnp.float32(1.0), jnp.float32(0.0))
        counts_acc[...] += onehot
        sse_acc[...] += dmin

    @pl.when(step == pl.num_programs(0) - 1)
    def _finish():
        counts = jnp.sum(counts_acc[...], axis=1, keepdims=True)
        p = counts / jnp.float32(N)
        ent = -jnp.sum(p * jnp.log(p + 1e-10))
        perp_ref[0, 0] = jnp.exp(ent)
        loss_ref[0, 0] = (1.0 + _COMMIT) * jnp.sum(sse_acc[...]) / jnp.float32(N * D)


_tc_call = pl.pallas_call(
    _tc_body,
    grid=(B // GRP,),
    in_specs=[
        pl.BlockSpec((GRP, D, T), lambda i: (i, 0, 0)),
        pl.BlockSpec((D, K), lambda i: (0, 0)),
    ],
    out_specs=[
        pl.BlockSpec((GRP, 1, TPAD), lambda i: (i, 0, 0)),
        pl.BlockSpec(memory_space=pltpu.SMEM, block_shape=(1, 1), index_map=lambda i: (0, 0)),
        pl.BlockSpec(memory_space=pltpu.SMEM, block_shape=(1, 1), index_map=lambda i: (0, 0)),
    ],
    out_shape=[
        jax.ShapeDtypeStruct((B, 1, TPAD), jnp.int32),
        jax.ShapeDtypeStruct((1, 1), jnp.float32),
        jax.ShapeDtypeStruct((1, 1), jnp.float32),
    ],
    scratch_shapes=[
        pltpu.VMEM((K, T), jnp.float32),
        pltpu.VMEM((K, 1), jnp.float32),
        pltpu.VMEM((1, T), jnp.float32),
    ],
)


@functools.lru_cache(maxsize=None)
def _make_sc_gather():
    @functools.partial(
        pl.kernel,
        mesh=plsc.VectorSubcoreMesh(core_axis_name="c", subcore_axis_name="s"),
        out_type=jax.ShapeDtypeStruct((B, T, D), jnp.float32),
        scratch_types=[
            pltpu.VMEM((_BPW,), jnp.int32),
            pltpu.VMEM((_BPW, D), jnp.float32),
            pltpu.SemaphoreType.DMA,
        ],
        compiler_params=pltpu.CompilerParams(use_tc_tiling_on_sc=False),
    )
    def _sc_gather(table_hbm, idx_hbm, out_hbm, idx_v, rows_v, sem):
        wid = lax.axis_index("s") * 2 + lax.axis_index("c")
        pltpu.sync_copy(idx_hbm.at[wid, 0, pl.ds(0, _BPW)], idx_v)
        copies = []
        for j in range(_NFULL):
            copies.append(
                pltpu.async_copy(
                    table_hbm.at[idx_v.at[pl.ds(j * _CHUNK, _CHUNK)]],
                    rows_v.at[pl.ds(j * _CHUNK, _CHUNK)],
                    sem,
                )
            )
        copies.append(
            pltpu.async_copy(
                table_hbm.at[idx_v.at[pl.ds(_NFULL * _CHUNK, _TAIL)]],
                rows_v.at[pl.ds(_NFULL * _CHUNK, _TAIL)],
                sem,
            )
        )
        for c in copies:
            c.wait()
        pltpu.sync_copy(rows_v, out_hbm.at[wid])

    return _sc_gather


def kernel(inputs, embeddings):
    xt = jnp.transpose(inputs, (0, 2, 1))
    idx, loss, perp = _tc_call(xt, embeddings)
    table = embeddings.T
    quantized_st = _make_sc_gather()(table, idx)
    return quantized_st, loss.reshape(()), perp.reshape(())


# speedup vs baseline: 1.0064x; 1.0043x over previous
import functools

import jax
import jax.numpy as jnp
from jax import lax
from jax.experimental import pallas as pl
from jax.experimental.pallas import tpu as pltpu
from jax.experimental.pallas import tpu_sc as plsc

D = 64
K = 1024
B = 32
T = 576
GRP = 16
TPAD = 640
N = B * T

_COMMIT = 0.25

_NW = 32
_BPW = N // _NW
_CHUNK = 128
_NFULL = _BPW // _CHUNK
_TAIL = _BPW - _NFULL * _CHUNK


def _tc_body(xt_ref, e_ref, idx_ref, loss_ref, perp_ref, counts_acc, e2col, sse_acc):
    step = pl.program_id(0)

    e = e_ref[...]

    @pl.when(step == 0)
    def _init():
        counts_acc[...] = jnp.zeros_like(counts_acc)
        sse_acc[...] = jnp.zeros_like(sse_acc)
        e2 = jnp.sum(e * e, axis=0, keepdims=True)
        e2col[...] = e2.reshape(K, 1)

    e2c = e2col[...]
    for g in range(GRP):
        xt = xt_ref[g]
        s2 = lax.dot_general(
            e, xt * jnp.float32(-2.0),
            (((0,), (0,)), ((), ())),
            preferred_element_type=jnp.float32,
        )
        x2 = jnp.sum(xt * xt, axis=0, keepdims=True)
        d = (x2 + s2) + e2c

        dmin = jnp.min(d, axis=0, keepdims=True)
        iota = lax.broadcasted_iota(jnp.int32, (K, T), 0).astype(jnp.float32)
        masked = jnp.where(d == dmin, iota, jnp.float32(K))
        idxf = jnp.min(masked, axis=0, keepdims=True)
        idx_ref[g, 0, pl.ds(0, T)] = idxf.astype(jnp.int32).reshape(T)

        onehot = jnp.where(masked == idxf, jnp.float32(1.0), jnp.float32(0.0))
        counts_acc[...] += onehot
        sse_acc[...] += dmin

    @pl.when(step == pl.num_programs(0) - 1)
    def _finish():
        counts = jnp.sum(counts_acc[...], axis=1, keepdims=True)
        p = counts / jnp.float32(N)
        ent = -jnp.sum(p * jnp.log(p + 1e-10))
        perp_ref[0, 0] = jnp.exp(ent)
        loss_ref[0, 0] = (1.0 + _COMMIT) * jnp.sum(sse_acc[...]) / jnp.float32(N * D)


_tc_call = pl.pallas_call(
    _tc_body,
    grid=(B // GRP,),
    in_specs=[
        pl.BlockSpec((GRP, D, T), lambda i: (i, 0, 0)),
        pl.BlockSpec((D, K), lambda i: (0, 0)),
    ],
    out_specs=[
        pl.BlockSpec((GRP, 1, TPAD), lambda i: (i, 0, 0)),
        pl.BlockSpec(memory_space=pltpu.SMEM, block_shape=(1, 1), index_map=lambda i: (0, 0)),
        pl.BlockSpec(memory_space=pltpu.SMEM, block_shape=(1, 1), index_map=lambda i: (0, 0)),
    ],
    out_shape=[
        jax.ShapeDtypeStruct((B, 1, TPAD), jnp.int32),
        jax.ShapeDtypeStruct((1, 1), jnp.float32),
        jax.ShapeDtypeStruct((1, 1), jnp.float32),
    ],
    scratch_shapes=[
        pltpu.VMEM((K, T), jnp.float32),
        pltpu.VMEM((K, 1), jnp.float32),
        pltpu.VMEM((1, T), jnp.float32),
    ],
)


@functools.lru_cache(maxsize=None)
def _make_sc_gather():
    @functools.partial(
        pl.kernel,
        mesh=plsc.VectorSubcoreMesh(core_axis_name="c", subcore_axis_name="s"),
        out_type=jax.ShapeDtypeStruct((B, T, D), jnp.float32),
        scratch_types=[
            pltpu.VMEM((_BPW,), jnp.int32),
            pltpu.VMEM((_BPW, D), jnp.float32),
            pltpu.SemaphoreType.DMA,
        ],
        compiler_params=pltpu.CompilerParams(use_tc_tiling_on_sc=False),
    )
    def _sc_gather(table_hbm, idx_hbm, out_hbm, idx_v, rows_v, sem):
        wid = lax.axis_index("s") * 2 + lax.axis_index("c")
        pltpu.sync_copy(idx_hbm.at[wid, 0, pl.ds(0, _BPW)], idx_v)
        copies = []
        for j in range(_NFULL):
            copies.append(
                pltpu.async_copy(
                    table_hbm.at[idx_v.at[pl.ds(j * _CHUNK, _CHUNK)]],
                    rows_v.at[pl.ds(j * _CHUNK, _CHUNK)],
                    sem,
                )
            )
        copies.append(
            pltpu.async_copy(
                table_hbm.at[idx_v.at[pl.ds(_NFULL * _CHUNK, _TAIL)]],
                rows_v.at[pl.ds(_NFULL * _CHUNK, _TAIL)],
                sem,
            )
        )
        for c in copies:
            c.wait()
        pltpu.sync_copy(rows_v, out_hbm.at[wid])

    return _sc_gather


def kernel(inputs, embeddings):
    xt = jnp.transpose(inputs, (0, 2, 1))
    idx, loss, perp = _tc_call(xt, embeddings)
    table = embeddings.T
    quantized_st = _make_sc_gather()(table, idx)
    return quantized_st, loss.reshape(()), perp.reshape(())
